# Initial kernel scaffold; baseline (speedup 1.0000x reference)
import jax
import jax.numpy as jnp
from jax.experimental import pallas as pl


def kernel(x, table, W, b):
    raise NotImplementedError("write your pallas kernel here")


# speedup vs baseline: 60.7017x; 60.7017x over previous
import functools

import jax
import jax.numpy as jnp
from jax import lax
from jax.experimental import pallas as pl
from jax.experimental.pallas import tpu as pltpu
from jax.experimental.pallas import tpu_sc as plsc

_VOCAB = 100000
_EMBED = 300
_ROWS_BLK = 4000
_LANES = 16
_NC = 2
_NS = 16
_NW = _NC * _NS
_CHUNK = 6400


def _proj_body(t_ref, w_ref, b_ref, o_ref):
    t = t_ref[...]
    w = w_ref[...]
    z = jnp.sum(t * w, axis=1, keepdims=True) + b_ref[0, 0]
    o_ref[...] = jax.nn.sigmoid(jnp.maximum(z, 0.0))


def _project_table(table, W, b):
    return pl.pallas_call(
        _proj_body,
        grid=(_VOCAB // _ROWS_BLK,),
        in_specs=[
            pl.BlockSpec((_ROWS_BLK, _EMBED), lambda i: (i, 0)),
            pl.BlockSpec((1, _EMBED), lambda i: (0, 0)),
            pl.BlockSpec((1, 1), lambda i: (0, 0)),
        ],
        out_specs=pl.BlockSpec((_ROWS_BLK, 1), lambda i: (i, 0)),
        out_shape=jax.ShapeDtypeStruct((_VOCAB, 1), jnp.float32),
    )(table, W, b.reshape(1, 1))


@functools.lru_cache(maxsize=None)
def _make_gather(total):
    per_w = total // _NW
    n_chunks = per_w // _CHUNK
    mesh = plsc.VectorSubcoreMesh(core_axis_name="c", subcore_axis_name="s")

    @functools.partial(
        pl.kernel,
        mesh=mesh,
        out_type=jax.ShapeDtypeStruct((total,), jnp.float32),
        scratch_types=[
            pltpu.VMEM((_VOCAB,), jnp.float32),
            pltpu.VMEM((_CHUNK,), jnp.int32),
            pltpu.VMEM((_CHUNK,), jnp.float32),
        ],
        compiler_params=pltpu.CompilerParams(needs_layout_passes=False),
    )
    def gather_k(s_hbm, idx_hbm, out_hbm, s_v, idx_v, out_v):
        wid = lax.axis_index("s") * _NC + lax.axis_index("c")
        base = wid * per_w
        pltpu.sync_copy(s_hbm, s_v)
        for c in range(n_chunks):
            off = base + c * _CHUNK
            pltpu.sync_copy(idx_hbm.at[pl.ds(off, _CHUNK)], idx_v)

            def body(i, _):
                idx16 = idx_v[pl.ds(i * _LANES, _LANES)]
                out_v[pl.ds(i * _LANES, _LANES)] = plsc.load_gather(
                    s_v, [idx16])
                return 0

            lax.fori_loop(0, _CHUNK // _LANES, body, 0)
            pltpu.sync_copy(out_v, out_hbm.at[pl.ds(off, _CHUNK)])

    return gather_k


def kernel(x, table, W, b):
    s = _project_table(table, W, b).reshape(_VOCAB)
    xf = x.reshape(-1)
    out = _make_gather(xf.size)(s, xf)
    return out.reshape(-1, 1)


# speedup vs baseline: 163.5547x; 2.6944x over previous
import functools

import jax
import jax.numpy as jnp
from jax import lax
from jax.experimental import pallas as pl
from jax.experimental.pallas import tpu as pltpu
from jax.experimental.pallas import tpu_sc as plsc

_VOCAB = 100000
_EMBED = 300
_COLS_BLK = 8192
_LANES = 16
_NC = 2
_NS = 16
_NW = _NC * _NS
_CHUNK = 6400


def _proj_body(t_ref, w_ref, b_ref, o_ref):
    t = t_ref[...]
    w = w_ref[...]
    z = jnp.sum(t * w, axis=0, keepdims=True) + b_ref[0, 0]
    o_ref[...] = jax.nn.sigmoid(jnp.maximum(z, 0.0))


def _project_table(table, W, b):
    return pl.pallas_call(
        _proj_body,
        grid=(pl.cdiv(_VOCAB, _COLS_BLK),),
        in_specs=[
            pl.BlockSpec((_EMBED, _COLS_BLK), lambda i: (0, i)),
            pl.BlockSpec((_EMBED, 1), lambda i: (0, 0)),
            pl.BlockSpec((1, 1), lambda i: (0, 0)),
        ],
        out_specs=pl.BlockSpec((1, _COLS_BLK), lambda i: (0, i)),
        out_shape=jax.ShapeDtypeStruct((1, _VOCAB), jnp.float32),
    )(table.T, W.reshape(_EMBED, 1), b.reshape(1, 1))


@functools.lru_cache(maxsize=None)
def _make_gather(total):
    per_w = total // _NW
    n_chunks = per_w // _CHUNK
    mesh = plsc.VectorSubcoreMesh(core_axis_name="c", subcore_axis_name="s")

    @functools.partial(
        pl.kernel,
        mesh=mesh,
        out_type=jax.ShapeDtypeStruct((total,), jnp.float32),
        scratch_types=[
            pltpu.VMEM((_VOCAB,), jnp.float32),
            pltpu.VMEM((_CHUNK,), jnp.int32),
            pltpu.VMEM((_CHUNK,), jnp.float32),
        ],
        compiler_params=pltpu.CompilerParams(needs_layout_passes=False),
    )
    def gather_k(s_hbm, idx_hbm, out_hbm, s_v, idx_v, out_v):
        wid = lax.axis_index("s") * _NC + lax.axis_index("c")
        base = wid * per_w
        pltpu.sync_copy(s_hbm, s_v)
        for c in range(n_chunks):
            off = base + c * _CHUNK
            pltpu.sync_copy(idx_hbm.at[pl.ds(off, _CHUNK)], idx_v)

            def body(i, _):
                idx16 = idx_v[pl.ds(i * _LANES, _LANES)]
                out_v[pl.ds(i * _LANES, _LANES)] = plsc.load_gather(
                    s_v, [idx16])
                return 0

            lax.fori_loop(0, _CHUNK // _LANES, body, 0)
            pltpu.sync_copy(out_v, out_hbm.at[pl.ds(off, _CHUNK)])

    return gather_k


def kernel(x, table, W, b):
    s = _project_table(table, W, b).reshape(_VOCAB)
    xf = x.reshape(-1)
    out = _make_gather(xf.size)(s, xf)
    return out.reshape(-1, 1)


# speedup vs baseline: 183.0796x; 1.1194x over previous
import functools

import jax
import jax.numpy as jnp
from jax import lax
from jax.experimental import pallas as pl
from jax.experimental.pallas import tpu as pltpu
from jax.experimental.pallas import tpu_sc as plsc

_VOCAB = 100000
_EMBED = 300
_COLS_BLK = 16384
_LANES = 16
_NC = 2
_NS = 16
_NW = _NC * _NS
_CHUNK = 6400


def _proj_body(t_ref, w_ref, b_ref, o_ref):
    t = t_ref[...]
    w = w_ref[...]
    z = jnp.sum(t * w, axis=0, keepdims=True) + b_ref[0, 0]
    o_ref[...] = jax.nn.sigmoid(jnp.maximum(z, 0.0))


def _project_table(table, W, b):
    return pl.pallas_call(
        _proj_body,
        grid=(pl.cdiv(_VOCAB, _COLS_BLK),),
        in_specs=[
            pl.BlockSpec((_EMBED, _COLS_BLK), lambda i: (0, i)),
            pl.BlockSpec((_EMBED, 1), lambda i: (0, 0)),
            pl.BlockSpec((1, 1), lambda i: (0, 0)),
        ],
        out_specs=pl.BlockSpec((1, _COLS_BLK), lambda i: (0, i)),
        out_shape=jax.ShapeDtypeStruct((1, _VOCAB), jnp.float32),
    )(table.T, W.reshape(_EMBED, 1), b.reshape(1, 1))


@functools.lru_cache(maxsize=None)
def _make_gather(total):
    per_w = total // _NW
    n_chunks = per_w // _CHUNK
    mesh = plsc.VectorSubcoreMesh(core_axis_name="c", subcore_axis_name="s")

    @functools.partial(
        pl.kernel,
        mesh=mesh,
        out_type=jax.ShapeDtypeStruct((total,), jnp.float32),
        scratch_types=[
            pltpu.VMEM((_VOCAB,), jnp.float32),
            pltpu.VMEM((2, _CHUNK), jnp.int32),
            pltpu.VMEM((2, _CHUNK), jnp.float32),
            pltpu.SemaphoreType.DMA,
            pltpu.SemaphoreType.DMA,
            pltpu.SemaphoreType.DMA,
            pltpu.SemaphoreType.DMA,
            pltpu.SemaphoreType.DMA,
        ],
        compiler_params=pltpu.CompilerParams(needs_layout_passes=False),
    )
    def gather_k(s_hbm, idx_hbm, out_hbm, s_v, idx_v, out_v,
                 sem_s, sem_i0, sem_i1, sem_o0, sem_o1):
        wid = lax.axis_index("s") * _NC + lax.axis_index("c")
        base = wid * per_w
        sem_i = (sem_i0, sem_i1)
        sem_o = (sem_o0, sem_o1)

        cp_s = pltpu.async_copy(s_hbm.at[0], s_v, sem_s)
        cp_i = {}
        for c in range(min(2, n_chunks)):
            cp_i[c] = pltpu.async_copy(
                idx_hbm.at[pl.ds(base + c * _CHUNK, _CHUNK)],
                idx_v.at[c % 2], sem_i[c % 2])
        cp_s.wait()

        cp_o = {}
        for c in range(n_chunks):
            sl = c % 2
            cp_i[c].wait()
            if c >= 2:
                cp_o[c - 2].wait()

            @plsc.parallel_loop(0, _CHUNK // _LANES, unroll=8)
            def _(i):
                idx16 = idx_v[sl, pl.ds(i * _LANES, _LANES)]
                out_v[sl, pl.ds(i * _LANES, _LANES)] = plsc.load_gather(
                    s_v, [idx16])

            cp_o[c] = pltpu.async_copy(
                out_v.at[sl], out_hbm.at[pl.ds(base + c * _CHUNK, _CHUNK)],
                sem_o[sl])
            if c + 2 < n_chunks:
                cp_i[c + 2] = pltpu.async_copy(
                    idx_hbm.at[pl.ds(base + (c + 2) * _CHUNK, _CHUNK)],
                    idx_v.at[sl], sem_i[sl])

        for c in range(max(0, n_chunks - 2), n_chunks):
            cp_o[c].wait()

    return gather_k


def kernel(x, table, W, b):
    s = _project_table(table, W, b)
    xf = x.reshape(-1)
    out = _make_gather(xf.size)(s, xf)
    return out.reshape(-1, 1)
